# Initial kernel scaffold; baseline (speedup 1.0000x reference)
import jax
import jax.numpy as jnp
from jax.experimental import pallas as pl


def kernel(x, edge_index, edge_attr, batch, W_i, W_h, W_o, b_o):
    raise NotImplementedError("write your pallas kernel here")


# speedup vs baseline: 2.3843x; 2.3843x over previous
import functools

import jax
import jax.numpy as jnp
from jax import lax
from jax.experimental import pallas as pl
from jax.experimental.pallas import tpu as pltpu
from jax.experimental.pallas import tpu_sc as plsc


F32 = jnp.float32


def _dot_nt(a, b):
    return lax.dot_general(a, b, (((1,), (1,)), ((), ())),
                           preferred_element_type=F32)


def _first_body(x_ref, wix_ref, wox_ref, bo_ref, h0_ref, xo_ref):
    xv = x_ref[...]
    h0_ref[...] = _dot_nt(xv, wix_ref[...])
    xo_ref[...] = _dot_nt(xv, wox_ref[...]) + bo_ref[...]


def _first_tc(x, W_ix, W_ox, b_o):
    n, f = x.shape
    h = W_ix.shape[0]
    return pl.pallas_call(
        _first_body,
        out_shape=(jax.ShapeDtypeStruct((n, h), F32),
                   jax.ShapeDtypeStruct((n, h), F32)),
    )(x, W_ix, W_ox, b_o.reshape(1, h))


def _edge_mm_body(ea_ref, w_ref, e_ref):
    e_ref[...] = _dot_nt(ea_ref[...], w_ref[...])


def _edge_mm(edge_attr, W_ie):
    e_total, bf = edge_attr.shape
    h = W_ie.shape[0]
    blk = 3200
    grid = e_total // blk
    return pl.pallas_call(
        _edge_mm_body,
        grid=(grid,),
        in_specs=[pl.BlockSpec((blk, bf), lambda i: (i, 0)),
                  pl.BlockSpec((h, bf), lambda i: (0, 0))],
        out_specs=pl.BlockSpec((blk, h), lambda i: (i, 0)),
        out_shape=jax.ShapeDtypeStruct((e_total, h), F32),
    )(edge_attr, W_ie)


def _sum_mm_body(np_ref, w_ref, o_ref):
    a = np_ref[0] + np_ref[1]
    o_ref[...] = _dot_nt(a, w_ref[...])


def _sum_mm(npart, W):
    _, n, h = npart.shape
    return pl.pallas_call(
        _sum_mm_body,
        out_shape=jax.ShapeDtypeStruct((n, h), F32),
    )(npart, W)


def _final_body(ngrid, ngraphs, xo_ref, np_ref, b_ref, w_ref, mol_ref, acc_ref):
    i = pl.program_id(0)
    nsum = np_ref[0] + np_ref[1]
    out = jnp.maximum(xo_ref[...] + _dot_nt(nsum, w_ref[...]), 0.0)
    bvals = b_ref[0, 0]
    gids = lax.broadcasted_iota(jnp.int32, (ngraphs, bvals.shape[0]), 0)
    onehot = (gids == bvals[None, :]).astype(F32)
    contrib = lax.dot_general(onehot, out, (((1,), (0,)), ((), ())),
                              preferred_element_type=F32)

    @pl.when(i == 0)
    def _():
        acc_ref[...] = jnp.zeros_like(acc_ref)

    acc_ref[...] += contrib
    mol_ref[...] = acc_ref[...]


def _final_tc(xo, npart, batch, W_oh, ngraphs):
    n, h = xo.shape
    blk = 2000
    ngrid = n // blk
    batch3 = batch.reshape(ngrid, 1, blk).astype(jnp.int32)
    return pl.pallas_call(
        functools.partial(_final_body, ngrid, ngraphs),
        grid=(ngrid,),
        in_specs=[pl.BlockSpec((blk, h), lambda i: (i, 0)),
                  pl.BlockSpec((2, blk, h), lambda i: (0, i, 0)),
                  pl.BlockSpec((1, 1, blk), lambda i: (i, 0, 0)),
                  pl.BlockSpec((h, h), lambda i: (0, 0))],
        out_specs=pl.BlockSpec((ngraphs, h), lambda i: (0, 0)),
        out_shape=jax.ShapeDtypeStruct((ngraphs, h), F32),
        scratch_shapes=[pltpu.VMEM((ngraphs, h), F32)],
    )(xo, npart, batch3, W_oh)


_NC = 2
_NS = 16
_B = 80
_ZR = 25


def _make_sc_pass(n, e, h, write_m):
    nw = _NC * _NS
    te = e // nw
    nblk = te // _B
    rpt = n // _NS
    assert te * nw == e and nblk * _B == te and rpt * _NS == n
    assert rpt % _ZR == 0
    hl = h // 16

    mesh = plsc.VectorSubcoreMesh(core_axis_name="c", subcore_axis_name="s")
    outs = [jax.ShapeDtypeStruct((nw, rpt, h), F32)]
    if write_m:
        outs = [jax.ShapeDtypeStruct((e, h), F32)] + outs

    @functools.partial(
        pl.kernel, mesh=mesh, out_type=tuple(outs),
        scratch_types=[
            pltpu.VMEM((_B,), jnp.int32),
            pltpu.VMEM((_B,), jnp.int32),
            pltpu.VMEM((_B, h), F32),
            pltpu.VMEM((_B, h), F32),
            pltpu.VMEM((_ZR, h), F32),
            pltpu.VMEM_SHARED((n, h), F32),
            pltpu.SemaphoreType.DMA,
        ],
    )
    def _pass(h_hbm, m_hbm, src_hbm, tgt_hbm, *rest):
        if write_m:
            mnew_hbm, npart_hbm = rest[0], rest[1]
            sidx, tidx, g_v, m_v, zb, acc, sem = rest[2:]
        else:
            npart_hbm = rest[0]
            sidx, tidx, g_v, m_v, zb, acc, sem = rest[1:]
        c = lax.axis_index("c")
        s = lax.axis_index("s")
        wid = s * _NC + c
        base = wid * te

        zero16 = jnp.zeros((16,), F32)

        @pl.loop(0, _ZR)
        def _(i):
            for j in range(hl):
                zb[i, pl.ds(j * 16, 16)] = zero16

        @pl.loop(0, rpt // _ZR)
        def _(k):
            pltpu.sync_copy(zb, acc.at[pl.ds(s * rpt + k * _ZR, _ZR)])

        plsc.subcore_barrier()

        @pl.loop(0, nblk)
        def _(blk):
            off = base + blk * _B
            pltpu.sync_copy(src_hbm.at[pl.ds(off, _B)], sidx)
            pltpu.sync_copy(tgt_hbm.at[pl.ds(off, _B)], tidx)
            pltpu.sync_copy(m_hbm.at[pl.ds(off, _B)], m_v)
            pltpu.async_copy(h_hbm.at[sidx], g_v, sem).wait()

            @pl.loop(0, _B)
            def _(i):
                for j in range(hl):
                    sl = pl.ds(j * 16, 16)
                    m_v[i, sl] = jnp.maximum(m_v[i, sl] + g_v[i, sl], 0.0)

            if write_m:
                pltpu.sync_copy(m_v, mnew_hbm.at[pl.ds(off, _B)])
            pltpu.sync_copy(m_v, acc.at[tidx], add=True)

        plsc.subcore_barrier()
        pltpu.sync_copy(acc.at[pl.ds(s * rpt, rpt)],
                        npart_hbm.at[c * _NS + s])

    return _pass


def kernel(x, edge_index, edge_attr, batch, W_i, W_h, W_o, b_o):
    n, f = x.shape
    e = edge_index.shape[1]
    h = W_i.shape[0]
    ngraphs = 512

    src = edge_index[0].astype(jnp.int32)
    tgt = edge_index[1].astype(jnp.int32)
    W_ix = W_i[:, :f]
    W_ie = W_i[:, f:]
    W_ox = W_o[:, :f]
    W_oh = W_o[:, f:]

    sc_pass_w = _make_sc_pass(n, e, h, write_m=True)
    sc_pass_n = _make_sc_pass(n, e, h, write_m=False)

    h0, xo = _first_tc(x, W_ix, W_ox, b_o)
    ev = _edge_mm(edge_attr, W_ie)

    m1, np1 = sc_pass_w(h0, ev, src, tgt)
    h1 = _sum_mm(np1.reshape(_NC, n, h), W_h)
    m2, np2 = sc_pass_w(h1, m1, src, tgt)
    h2 = _sum_mm(np2.reshape(_NC, n, h), W_h)
    (np3,) = sc_pass_n(h2, m2, src, tgt)

    return _final_tc(xo, np3.reshape(_NC, n, h), batch, W_oh, ngraphs)


# speedup vs baseline: 5.1317x; 2.1523x over previous
import functools

import jax
import jax.numpy as jnp
from jax import lax
from jax.experimental import pallas as pl
from jax.experimental.pallas import tpu as pltpu
from jax.experimental.pallas import tpu_sc as plsc


F32 = jnp.float32


def _dot_nt(a, b):
    return lax.dot_general(a, b, (((1,), (1,)), ((), ())),
                           preferred_element_type=F32)


def _first_body(x_ref, wix_ref, wox_ref, bo_ref, h0_ref, xo_ref):
    xv = x_ref[...]
    h0_ref[...] = _dot_nt(xv, wix_ref[...])
    xo_ref[...] = _dot_nt(xv, wox_ref[...]) + bo_ref[...]


def _first_tc(x, W_ix, W_ox, b_o):
    n, f = x.shape
    h = W_ix.shape[0]
    return pl.pallas_call(
        _first_body,
        out_shape=(jax.ShapeDtypeStruct((n, h), F32),
                   jax.ShapeDtypeStruct((n, h), F32)),
    )(x, W_ix, W_ox, b_o.reshape(1, h))


def _edge_mm_body(ea_ref, w_ref, e_ref):
    e_ref[...] = _dot_nt(ea_ref[...], w_ref[...])


def _edge_mm(edge_attr, W_ie):
    e_total, bf = edge_attr.shape
    h = W_ie.shape[0]
    blk = 3200
    grid = e_total // blk
    return pl.pallas_call(
        _edge_mm_body,
        grid=(grid,),
        in_specs=[pl.BlockSpec((blk, bf), lambda i: (i, 0)),
                  pl.BlockSpec((h, bf), lambda i: (0, 0))],
        out_specs=pl.BlockSpec((blk, h), lambda i: (i, 0)),
        out_shape=jax.ShapeDtypeStruct((e_total, h), F32),
    )(edge_attr, W_ie)


def _sum_mm_body(np_ref, w_ref, o_ref):
    a = np_ref[0] + np_ref[1]
    o_ref[...] = _dot_nt(a, w_ref[...])


def _sum_mm(npart, W):
    _, n, h = npart.shape
    return pl.pallas_call(
        _sum_mm_body,
        out_shape=jax.ShapeDtypeStruct((n, h), F32),
    )(npart, W)


def _final_body(ngraphs, xo_ref, np_ref, b_ref, w_ref, mol_ref, acc_ref):
    i = pl.program_id(0)
    nsum = np_ref[0] + np_ref[1]
    out = jnp.maximum(xo_ref[...] + _dot_nt(nsum, w_ref[...]), 0.0)
    bvals = b_ref[0, 0]
    gids = lax.broadcasted_iota(jnp.int32, (ngraphs, bvals.shape[0]), 0)
    onehot = (gids == bvals[None, :]).astype(F32)
    contrib = lax.dot_general(onehot, out, (((1,), (0,)), ((), ())),
                              preferred_element_type=F32)

    @pl.when(i == 0)
    def _():
        acc_ref[...] = jnp.zeros_like(acc_ref)

    acc_ref[...] += contrib
    mol_ref[...] = acc_ref[...]


def _final_tc(xo, npart, batch, W_oh, ngraphs):
    n, h = xo.shape
    blk = 2000
    ngrid = n // blk
    batch3 = batch.reshape(ngrid, 1, blk).astype(jnp.int32)
    return pl.pallas_call(
        functools.partial(_final_body, ngraphs),
        grid=(ngrid,),
        in_specs=[pl.BlockSpec((blk, h), lambda i: (i, 0)),
                  pl.BlockSpec((2, blk, h), lambda i: (0, i, 0)),
                  pl.BlockSpec((1, 1, blk), lambda i: (i, 0, 0)),
                  pl.BlockSpec((h, h), lambda i: (0, 0))],
        out_specs=pl.BlockSpec((ngraphs, h), lambda i: (0, 0)),
        out_shape=jax.ShapeDtypeStruct((ngraphs, h), F32),
        scratch_shapes=[pltpu.VMEM((ngraphs, h), F32)],
    )(xo, npart, batch3, W_oh)


_NC = 2
_NS = 16
_B = 40
_ZR = 25


def _make_sc_pass(n, e, h, write_m):
    nw = _NC * _NS
    te = e // nw
    nblk = te // _B
    rpt = n // _NS
    assert te * nw == e and nblk * _B == te and rpt * _NS == n
    assert rpt % _ZR == 0 and nblk % 2 == 0 and _ZR <= _B
    hl = h // 16

    mesh = plsc.VectorSubcoreMesh(core_axis_name="c", subcore_axis_name="s")
    outs = [jax.ShapeDtypeStruct((nw, rpt, h), F32)]
    if write_m:
        outs = [jax.ShapeDtypeStruct((e, h), F32)] + outs

    @functools.partial(
        pl.kernel, mesh=mesh, out_type=tuple(outs),
        scratch_types=[
            pltpu.VMEM((2, _B), jnp.int32),
            pltpu.VMEM((te,), jnp.int32),
            pltpu.VMEM((2, _B, h), F32),
            pltpu.VMEM((2, _B, h), F32),
            pltpu.VMEM((2, _B, h), F32),
            pltpu.SemaphoreType.DMA((2,)),
            pltpu.SemaphoreType.DMA((2,)),
            pltpu.SemaphoreType.DMA((2,)),
            pltpu.SemaphoreType.DMA((2,)),
            pltpu.SemaphoreType.DMA((2,)),
            pltpu.VMEM_SHARED((n, h), F32),
        ],
    )
    def _pass(h_hbm, m_hbm, src_hbm, tgt_hbm, *rest):
        if write_m:
            mnew_hbm, npart_hbm = rest[0], rest[1]
            rest = rest[2:]
        else:
            npart_hbm = rest[0]
            rest = rest[1:]
        (tidx, sidx, g_v, l_v, o_v,
         tsem, gsem, msem, wlsem, wssem, acc) = rest
        c = lax.axis_index("c")
        s = lax.axis_index("s")
        wid = s * _NC + c
        base = wid * te

        def m_rows(k):
            return pl.ds(pl.multiple_of(base + k * _B, 8), _B)

        def e_rows(k):
            return pl.ds(pl.multiple_of(k * _B, 8), _B)

        def issue_tgt(k, j):
            pltpu.async_copy(tgt_hbm.at[m_rows(k)], tidx.at[j], tsem.at[j])

        def wait_tgt(k, j):
            pltpu.make_async_copy(tgt_hbm.at[m_rows(k)], tidx.at[j],
                                  tsem.at[j]).wait()

        def issue_loads(k, j):
            pltpu.async_copy(h_hbm.at[sidx.at[e_rows(k)]], g_v.at[j],
                             gsem.at[j])
            pltpu.async_copy(m_hbm.at[m_rows(k)], l_v.at[j], msem.at[j])

        def wait_loads(k, j):
            pltpu.make_async_copy(h_hbm.at[sidx.at[e_rows(k)]], g_v.at[j],
                                  gsem.at[j]).wait()
            pltpu.make_async_copy(m_hbm.at[m_rows(k)], l_v.at[j],
                                  msem.at[j]).wait()

        def issue_writes(k, j):
            if write_m:
                pltpu.async_copy(o_v.at[j], mnew_hbm.at[m_rows(k)],
                                 wlsem.at[j])
            pltpu.async_copy(o_v.at[j], acc.at[tidx.at[j]], wssem.at[j],
                             add=True)

        def wait_writes(k, j):
            if write_m:
                pltpu.make_async_copy(o_v.at[j], mnew_hbm.at[m_rows(k)],
                                      wlsem.at[j]).wait()
            pltpu.make_async_copy(o_v.at[j], acc.at[tidx.at[j]],
                                  wssem.at[j]).wait()

        zero16 = jnp.zeros((16,), F32)

        @pl.loop(0, _ZR)
        def _(i):
            for j in range(hl):
                o_v[0, i, pl.ds(j * 16, 16)] = zero16

        zsrc = o_v.at[0].at[pl.ds(0, _ZR)]

        @pl.loop(0, rpt // _ZR)
        def _(k):
            pltpu.sync_copy(zsrc, acc.at[pl.ds(s * rpt + k * _ZR, _ZR)])

        plsc.subcore_barrier()

        pltpu.sync_copy(src_hbm.at[pl.ds(pl.multiple_of(base, 8), te)], sidx)
        issue_tgt(0, 0)
        issue_tgt(1, 1)
        issue_loads(0, 0)
        issue_loads(1, 1)

        @pl.loop(0, nblk // 2)
        def _(t):
            k0 = t * 2
            for p in range(2):
                k = k0 + p
                q = 1 - p
                wait_loads(k, p)

                @pl.loop(0, _B)
                def _(r):
                    for j in range(hl):
                        sl = pl.ds(j * 16, 16)
                        o_v[p, r, sl] = jnp.maximum(
                            l_v[p, r, sl] + g_v[p, r, sl], 0.0)

                wait_tgt(k, p)
                issue_writes(k, p)

                @pl.when(k >= 1)
                def _():
                    wait_writes(k - 1, q)

                @pl.when(jnp.logical_and(k >= 1, k + 1 < nblk))
                def _():
                    issue_tgt(k + 1, q)

                @pl.when(k + 2 < nblk)
                def _():
                    issue_loads(k + 2, p)

        wait_writes(nblk - 1, (nblk - 1) % 2)
        plsc.subcore_barrier()
        pltpu.sync_copy(acc.at[pl.ds(s * rpt, rpt)],
                        npart_hbm.at[c * _NS + s])

    return _pass


def kernel(x, edge_index, edge_attr, batch, W_i, W_h, W_o, b_o):
    n, f = x.shape
    e = edge_index.shape[1]
    h = W_i.shape[0]
    ngraphs = 512
    nw = _NC * _NS

    src = edge_index[0].astype(jnp.int32)
    tgt = edge_index[1].astype(jnp.int32)
    W_ix = W_i[:, :f]
    W_ie = W_i[:, f:]
    W_ox = W_o[:, :f]
    W_oh = W_o[:, f:]

    sc_pass_w = _make_sc_pass(n, e, h, write_m=True)
    sc_pass_n = _make_sc_pass(n, e, h, write_m=False)

    h0, xo = _first_tc(x, W_ix, W_ox, b_o)
    ev = _edge_mm(edge_attr, W_ie)

    m1, np1 = sc_pass_w(h0, ev, src, tgt)
    h1 = _sum_mm(np1.reshape(_NC, n, h), W_h)
    m2, np2 = sc_pass_w(h1, m1, src, tgt)
    h2 = _sum_mm(np2.reshape(_NC, n, h), W_h)
    (np3,) = sc_pass_n(h2, m2, src, tgt)

    return _final_tc(xo, np3.reshape(_NC, n, h), batch, W_oh, ngraphs)
